# baseline (speedup 1.0000x reference)
import functools

import jax
import jax.numpy as jnp
from jax import lax
from jax.experimental import pallas as pl
from jax.experimental.pallas import tpu as pltpu
from jax.experimental.pallas import tpu_sc as plsc

H = 8
K = 1024
D = 256
N = 8192
BETA = 0.25

BN = 512
S = 4
NSEG = N // S
NB = NSEG // BN
ROWS_SEG = NSEG * H

NC = 2
NS = 16
NW = NC * NS
B_PER_W = ROWS_SEG // NW
C = 128
NCHUNK = B_PER_W // C

_LOSS_SCALE = (1.0 + BETA) / (N * D * H)


def _dist_kernel(e_ref, cb_ref, idx_ref, gidx_ref, loss_ref, cbn_ref):
    nb = pl.program_id(0)

    @pl.when(nb == 0)
    def _init():
        loss_ref[0, 0] = 0.0
        for h in range(H):
            cb = cb_ref[h]
            cbn_ref[h] = jnp.sum(cb * cb, axis=1)

    iota_f = lax.broadcasted_iota(jnp.int32, (BN, K), 1).astype(jnp.float32)
    qs = []
    gqs = []
    block_loss = 0.0
    for h in range(H):
        z = e_ref[:, h * D:(h + 1) * D]
        cb = cb_ref[h]
        zc = lax.dot_general(z, cb, (((1,), (1,)), ((), ())),
                             preferred_element_type=jnp.float32)
        rn = jnp.sum(z * z, axis=1, keepdims=True)
        d = (rn - 2.0 * zc) + cbn_ref[h][None, :]
        mn = jnp.min(d, axis=1, keepdims=True)
        qf = jnp.min(jnp.where(d == mn, iota_f, float(K)), axis=1)
        q = qf.astype(jnp.int32)
        qs.append(q)
        gqs.append(q + h * K)
        block_loss = block_loss + jnp.sum(mn)

    idx_ref[0] = jnp.stack(qs, axis=1)
    gidx_ref[0] = jnp.stack(gqs, axis=1)
    loss_ref[0, 0] += block_loss

    @pl.when(nb == NB - 1)
    def _fin():
        loss_ref[0, 0] = loss_ref[0, 0] * _LOSS_SCALE


def _distances(eseg, codebooks):
    return pl.pallas_call(
        _dist_kernel,
        grid=(NB,),
        in_specs=[
            pl.BlockSpec((BN, H * D), lambda nb: (nb, 0)),
            pl.BlockSpec((H, K, D), lambda nb: (0, 0, 0)),
        ],
        out_specs=[
            pl.BlockSpec((1, BN, H), lambda nb: (nb, 0, 0)),
            pl.BlockSpec((1, BN, H), lambda nb: (nb, 0, 0)),
            pl.BlockSpec(memory_space=pltpu.SMEM),
        ],
        out_shape=[
            jax.ShapeDtypeStruct((NB, BN, H), jnp.int32),
            jax.ShapeDtypeStruct((NB, BN, H), jnp.int32),
            jax.ShapeDtypeStruct((1, 1), jnp.float32),
        ],
        scratch_shapes=[pltpu.VMEM((H, K), jnp.float32)],
    )(eseg, codebooks)


@functools.lru_cache(maxsize=None)
def _make_gather():
    mesh = plsc.VectorSubcoreMesh(core_axis_name="c", subcore_axis_name="s")

    @functools.partial(
        pl.kernel,
        mesh=mesh,
        out_type=jax.ShapeDtypeStruct((ROWS_SEG, D), jnp.float32),
        scratch_types=[
            pltpu.VMEM((NCHUNK, C), jnp.int32),
            pltpu.VMEM((3, C, D), jnp.float32),
            pltpu.SemaphoreType.DMA,
            pltpu.SemaphoreType.DMA,
        ],
    )
    def _gather(table_hbm, gidx_hbm, out_hbm, idx_v, rows_v, gsem, ssem):
        wid = lax.axis_index("s") * NC + lax.axis_index("c")
        base = wid * B_PER_W
        pltpu.sync_copy(gidx_hbm.at[wid], idx_v)
        gcp = [None] * NCHUNK
        scp = [None] * NCHUNK
        for c in range(min(2, NCHUNK)):
            gcp[c] = pltpu.async_copy(
                table_hbm.at[idx_v.at[c]], rows_v.at[c % 3], gsem)
        for c in range(NCHUNK):
            gcp[c].wait()
            scp[c] = pltpu.async_copy(
                rows_v.at[c % 3], out_hbm.at[pl.ds(base + c * C, C)], ssem)
            nxt = c + 2
            if nxt < NCHUNK:
                if nxt - 3 >= 0:
                    scp[nxt - 3].wait()
                gcp[nxt] = pltpu.async_copy(
                    table_hbm.at[idx_v.at[nxt]], rows_v.at[nxt % 3], gsem)
        for c in range(max(0, NCHUNK - 3), NCHUNK):
            scp[c].wait()

    return _gather


def kernel(embeds, codebooks):
    table = codebooks.reshape(H * K, D)
    gather = _make_gather()
    qbuf = jnp.zeros((N * H, D), jnp.float32)
    idx_parts = []
    loss = jnp.float32(0.0)
    for s in range(S):
        eseg = embeds[s * NSEG:(s + 1) * NSEG].reshape(NSEG, H * D)
        idx3, gidx3, lseg = _distances(eseg, codebooks)
        qpart = gather(table, gidx3.reshape(NW, NCHUNK, C))
        qbuf = lax.dynamic_update_slice(qbuf, qpart, (s * ROWS_SEG, 0))
        idx_parts.append(idx3.reshape(NSEG, H))
        loss = loss + lseg[0, 0]
    indices = jnp.concatenate(idx_parts, axis=0)
    quantized = qbuf.reshape(N, H, D)
    return quantized, indices, loss


# speedup vs baseline: 1.3323x; 1.3323x over previous
import functools

import jax
import jax.numpy as jnp
from jax import lax
from jax.experimental import pallas as pl
from jax.experimental.pallas import tpu as pltpu
from jax.experimental.pallas import tpu_sc as plsc

H = 8
K = 1024
D = 256
N = 8192
BETA = 0.25

BN = 512
NB = N // BN

NC = 2
NS = 16
NW = NC * NS
BFLAT = N * H
B_PER_W = BFLAT // NW
C = 128
NCHUNK = B_PER_W // C

_LOSS_SCALE = (1.0 + BETA) / (N * D * H)


def _dist_kernel(e_ref, cb_ref, idx_ref, gidx_ref, loss_ref, cbn_ref):
    nb = pl.program_id(0)

    @pl.when(nb == 0)
    def _init():
        loss_ref[0, 0] = 0.0
        for h in range(H):
            cb = cb_ref[h]
            cbn_ref[h] = jnp.sum(cb * cb, axis=1)

    et = jnp.swapaxes(e_ref[...], 0, 1)
    iota_f = lax.broadcasted_iota(jnp.int32, (BN, K), 1).astype(jnp.float32)
    qs = []
    gqs = []
    block_loss = 0.0
    for h in range(H):
        z = et[h]
        cb = cb_ref[h]
        zc = lax.dot_general(z, cb, (((1,), (1,)), ((), ())),
                             preferred_element_type=jnp.float32)
        rn = jnp.sum(z * z, axis=1, keepdims=True)
        d = (rn - 2.0 * zc) + cbn_ref[h][None, :]
        mn = jnp.min(d, axis=1, keepdims=True)
        qf = jnp.min(jnp.where(d == mn, iota_f, float(K)), axis=1)
        q = qf.astype(jnp.int32)
        qs.append(q)
        gqs.append(q + h * K)
        block_loss = block_loss + jnp.sum(mn)

    idx_ref[0] = jnp.stack(qs, axis=1)
    gidx_ref[0] = jnp.stack(gqs, axis=1)
    loss_ref[0, 0] += block_loss

    @pl.when(nb == NB - 1)
    def _fin():
        loss_ref[0, 0] = loss_ref[0, 0] * _LOSS_SCALE


def _distances(embeds, codebooks):
    return pl.pallas_call(
        _dist_kernel,
        grid=(NB,),
        in_specs=[
            pl.BlockSpec((BN, H, D), lambda nb: (nb, 0, 0)),
            pl.BlockSpec((H, K, D), lambda nb: (0, 0, 0)),
        ],
        out_specs=[
            pl.BlockSpec((1, BN, H), lambda nb: (nb, 0, 0)),
            pl.BlockSpec((1, BN, H), lambda nb: (nb, 0, 0)),
            pl.BlockSpec(memory_space=pltpu.SMEM),
        ],
        out_shape=[
            jax.ShapeDtypeStruct((NB, BN, H), jnp.int32),
            jax.ShapeDtypeStruct((NB, BN, H), jnp.int32),
            jax.ShapeDtypeStruct((1, 1), jnp.float32),
        ],
        scratch_shapes=[pltpu.VMEM((H, K), jnp.float32)],
    )(embeds, codebooks)


@functools.lru_cache(maxsize=None)
def _make_gather():
    mesh = plsc.VectorSubcoreMesh(core_axis_name="c", subcore_axis_name="s")

    @functools.partial(
        pl.kernel,
        mesh=mesh,
        out_type=jax.ShapeDtypeStruct((BFLAT, D), jnp.float32),
        scratch_types=[
            pltpu.VMEM((NCHUNK, C), jnp.int32),
            pltpu.VMEM((3, C, D), jnp.float32),
            pltpu.SemaphoreType.DMA,
            pltpu.SemaphoreType.DMA,
        ],
    )
    def _gather(table_hbm, gidx_hbm, out_hbm, idx_v, rows_v, gsem, ssem):
        wid = lax.axis_index("s") * NC + lax.axis_index("c")
        base = wid * B_PER_W
        pltpu.sync_copy(gidx_hbm.at[wid], idx_v)
        gcp = [None] * NCHUNK
        scp = [None] * NCHUNK
        for c in range(min(2, NCHUNK)):
            gcp[c] = pltpu.async_copy(
                table_hbm.at[idx_v.at[c]], rows_v.at[c % 3], gsem)
        for c in range(NCHUNK):
            gcp[c].wait()
            scp[c] = pltpu.async_copy(
                rows_v.at[c % 3], out_hbm.at[pl.ds(base + c * C, C)], ssem)
            nxt = c + 2
            if nxt < NCHUNK:
                if nxt - 3 >= 0:
                    scp[nxt - 3].wait()
                gcp[nxt] = pltpu.async_copy(
                    table_hbm.at[idx_v.at[nxt]], rows_v.at[nxt % 3], gsem)
        for c in range(max(0, NCHUNK - 3), NCHUNK):
            scp[c].wait()

    return _gather


def kernel(embeds, codebooks):
    idx3, gidx3, loss = _distances(embeds, codebooks)
    indices = idx3.reshape(N, H)
    gflat = gidx3.reshape(NW, NCHUNK, C)
    table = codebooks.reshape(H * K, D)
    qflat = _make_gather()(table, gflat)
    quantized = qflat.reshape(N, H, D)
    return quantized, indices, loss[0, 0]


# speedup vs baseline: 1.4349x; 1.0770x over previous
import functools

import jax
import jax.numpy as jnp
from jax import lax
from jax.experimental import pallas as pl
from jax.experimental.pallas import tpu as pltpu
from jax.experimental.pallas import tpu_sc as plsc

H = 8
K = 1024
D = 256
N = 8192
BETA = 0.25

BN = 512
NB = N // BN

NC = 2
NS = 16
NW = NC * NS
BFLAT = N * H
B_PER_W = BFLAT // NW
C = 128
NCHUNK = B_PER_W // C

_LOSS_SCALE = (1.0 + BETA) / (N * D * H)


def _dist_kernel(e_hbm, cb_ref, idx_ref, gidx_ref, loss_ref,
                 cbn_ref, z_scr, sem):
    nb = pl.program_id(0)

    @pl.when(nb == 0)
    def _init():
        loss_ref[0, 0] = 0.0
        for h in range(H):
            cb = cb_ref[h]
            cbn_ref[h] = jnp.sum(cb * cb, axis=1)

    def _copy(nbb, slot, h):
        return pltpu.make_async_copy(
            e_hbm.at[pl.ds(nbb * BN, BN), h, :],
            z_scr.at[slot, h],
            sem.at[slot])

    slot = lax.rem(nb, 2)

    @pl.when(nb == 0)
    def _prologue():
        for h in range(H):
            _copy(nb, slot, h).start()

    @pl.when(nb + 1 < NB)
    def _prefetch():
        for h in range(H):
            _copy(nb + 1, 1 - slot, h).start()

    for h in range(H):
        _copy(nb, slot, h).wait()

    iota_f = lax.broadcasted_iota(jnp.int32, (BN, K), 1).astype(jnp.float32)
    qs = []
    gqs = []
    block_loss = 0.0
    for h in range(H):
        z = z_scr[slot, h]
        cb = cb_ref[h]
        zc = lax.dot_general(z, cb, (((1,), (1,)), ((), ())),
                             preferred_element_type=jnp.float32)
        rn = jnp.sum(z * z, axis=1, keepdims=True)
        d = (rn - 2.0 * zc) + cbn_ref[h][None, :]
        mn = jnp.min(d, axis=1, keepdims=True)
        qf = jnp.min(jnp.where(d == mn, iota_f, float(K)), axis=1)
        q = qf.astype(jnp.int32)
        qs.append(q)
        gqs.append(q + h * K)
        block_loss = block_loss + jnp.sum(mn)

    idx_ref[0] = jnp.stack(qs, axis=1)
    gidx_ref[0] = jnp.stack(gqs, axis=1)
    loss_ref[0, 0] += block_loss

    @pl.when(nb == NB - 1)
    def _fin():
        loss_ref[0, 0] = loss_ref[0, 0] * _LOSS_SCALE


def _distances(embeds, codebooks):
    return pl.pallas_call(
        _dist_kernel,
        grid=(NB,),
        in_specs=[
            pl.BlockSpec(memory_space=pl.ANY),
            pl.BlockSpec((H, K, D), lambda nb: (0, 0, 0)),
        ],
        out_specs=[
            pl.BlockSpec((1, BN, H), lambda nb: (nb, 0, 0)),
            pl.BlockSpec((1, BN, H), lambda nb: (nb, 0, 0)),
            pl.BlockSpec(memory_space=pltpu.SMEM),
        ],
        out_shape=[
            jax.ShapeDtypeStruct((NB, BN, H), jnp.int32),
            jax.ShapeDtypeStruct((NB, BN, H), jnp.int32),
            jax.ShapeDtypeStruct((1, 1), jnp.float32),
        ],
        scratch_shapes=[
            pltpu.VMEM((H, K), jnp.float32),
            pltpu.VMEM((2, H, BN, D), jnp.float32),
            pltpu.SemaphoreType.DMA((2,)),
        ],
    )(embeds, codebooks)


@functools.lru_cache(maxsize=None)
def _make_gather():
    mesh = plsc.VectorSubcoreMesh(core_axis_name="c", subcore_axis_name="s")

    @functools.partial(
        pl.kernel,
        mesh=mesh,
        out_type=jax.ShapeDtypeStruct((BFLAT, D), jnp.float32),
        scratch_types=[
            pltpu.VMEM((NCHUNK, C), jnp.int32),
            pltpu.VMEM((3, C, D), jnp.float32),
            pltpu.SemaphoreType.DMA,
            pltpu.SemaphoreType.DMA,
        ],
    )
    def _gather(table_hbm, gidx_hbm, out_hbm, idx_v, rows_v, gsem, ssem):
        wid = lax.axis_index("s") * NC + lax.axis_index("c")
        base = wid * B_PER_W
        pltpu.sync_copy(gidx_hbm.at[wid], idx_v)
        gcp = [None] * NCHUNK
        scp = [None] * NCHUNK
        for c in range(min(2, NCHUNK)):
            gcp[c] = pltpu.async_copy(
                table_hbm.at[idx_v.at[c]], rows_v.at[c % 3], gsem)
        for c in range(NCHUNK):
            gcp[c].wait()
            scp[c] = pltpu.async_copy(
                rows_v.at[c % 3], out_hbm.at[pl.ds(base + c * C, C)], ssem)
            nxt = c + 2
            if nxt < NCHUNK:
                if nxt - 3 >= 0:
                    scp[nxt - 3].wait()
                gcp[nxt] = pltpu.async_copy(
                    table_hbm.at[idx_v.at[nxt]], rows_v.at[nxt % 3], gsem)
        for c in range(max(0, NCHUNK - 3), NCHUNK):
            scp[c].wait()

    return _gather


def kernel(embeds, codebooks):
    idx3, gidx3, loss = _distances(embeds, codebooks)
    indices = idx3.reshape(N, H)
    gflat = gidx3.reshape(NW, NCHUNK, C)
    table = codebooks.reshape(H * K, D)
    qflat = _make_gather()(table, gflat)
    quantized = qflat.reshape(N, H, D)
    return quantized, indices, loss[0, 0]


# speedup vs baseline: 1.4756x; 1.0284x over previous
import functools

import jax
import jax.numpy as jnp
from jax import lax
from jax.experimental import pallas as pl
from jax.experimental.pallas import tpu as pltpu
from jax.experimental.pallas import tpu_sc as plsc

H = 8
K = 1024
D = 256
N = 8192
BETA = 0.25

BN = 1024
NB = N // BN

NC = 2
NS = 16
NW = NC * NS
BFLAT = N * H
B_PER_W = BFLAT // NW
C = 128
NCHUNK = B_PER_W // C

_LOSS_SCALE = (1.0 + BETA) / (N * D * H)


def _dist_kernel(e_hbm, cb_ref, idx_ref, gidx_ref, loss_ref,
                 cbn_ref, z_scr, sem):
    nb = pl.program_id(0)

    @pl.when(nb == 0)
    def _init():
        loss_ref[0, 0] = 0.0
        for h in range(H):
            cb = cb_ref[h]
            cbn_ref[h] = jnp.sum(cb * cb, axis=1)

    def _copy(nbb, slot, h):
        return pltpu.make_async_copy(
            e_hbm.at[pl.ds(nbb * BN, BN), h, :],
            z_scr.at[slot, h],
            sem.at[slot])

    slot = lax.rem(nb, 2)

    @pl.when(nb == 0)
    def _prologue():
        for h in range(H):
            _copy(nb, slot, h).start()

    @pl.when(nb + 1 < NB)
    def _prefetch():
        for h in range(H):
            _copy(nb + 1, 1 - slot, h).start()

    for h in range(H):
        _copy(nb, slot, h).wait()

    iota_f = lax.broadcasted_iota(jnp.int32, (BN, K), 1).astype(jnp.float32)
    qs = []
    gqs = []
    block_loss = 0.0
    for h in range(H):
        z = z_scr[slot, h]
        cb = cb_ref[h]
        zc = lax.dot_general(z, cb, (((1,), (1,)), ((), ())),
                             preferred_element_type=jnp.float32)
        rn = jnp.sum(z * z, axis=1, keepdims=True)
        d = (rn - 2.0 * zc) + cbn_ref[h][None, :]
        mn = jnp.min(d, axis=1, keepdims=True)
        qf = jnp.min(jnp.where(d == mn, iota_f, float(K)), axis=1)
        q = qf.astype(jnp.int32)
        qs.append(q)
        gqs.append(q + h * K)
        block_loss = block_loss + jnp.sum(mn)

    idx_ref[0] = jnp.stack(qs, axis=1)
    gidx_ref[0] = jnp.stack(gqs, axis=1)
    loss_ref[0, 0] += block_loss

    @pl.when(nb == NB - 1)
    def _fin():
        loss_ref[0, 0] = loss_ref[0, 0] * _LOSS_SCALE


def _distances(embeds, codebooks):
    return pl.pallas_call(
        _dist_kernel,
        grid=(NB,),
        in_specs=[
            pl.BlockSpec(memory_space=pl.ANY),
            pl.BlockSpec((H, K, D), lambda nb: (0, 0, 0)),
        ],
        out_specs=[
            pl.BlockSpec((1, BN, H), lambda nb: (nb, 0, 0)),
            pl.BlockSpec((1, BN, H), lambda nb: (nb, 0, 0)),
            pl.BlockSpec(memory_space=pltpu.SMEM),
        ],
        out_shape=[
            jax.ShapeDtypeStruct((NB, BN, H), jnp.int32),
            jax.ShapeDtypeStruct((NB, BN, H), jnp.int32),
            jax.ShapeDtypeStruct((1, 1), jnp.float32),
        ],
        scratch_shapes=[
            pltpu.VMEM((H, K), jnp.float32),
            pltpu.VMEM((2, H, BN, D), jnp.float32),
            pltpu.SemaphoreType.DMA((2,)),
        ],
    )(embeds, codebooks)


@functools.lru_cache(maxsize=None)
def _make_gather():
    mesh = plsc.VectorSubcoreMesh(core_axis_name="c", subcore_axis_name="s")

    @functools.partial(
        pl.kernel,
        mesh=mesh,
        out_type=jax.ShapeDtypeStruct((BFLAT, D), jnp.float32),
        scratch_types=[
            pltpu.VMEM((NCHUNK, C), jnp.int32),
            pltpu.VMEM((3, C, D), jnp.float32),
            pltpu.SemaphoreType.DMA,
            pltpu.SemaphoreType.DMA,
        ],
    )
    def _gather(table_hbm, gidx_hbm, out_hbm, idx_v, rows_v, gsem, ssem):
        wid = lax.axis_index("s") * NC + lax.axis_index("c")
        base = wid * B_PER_W
        pltpu.sync_copy(gidx_hbm.at[wid], idx_v)
        gcp = [None] * NCHUNK
        scp = [None] * NCHUNK
        for c in range(min(2, NCHUNK)):
            gcp[c] = pltpu.async_copy(
                table_hbm.at[idx_v.at[c]], rows_v.at[c % 3], gsem)
        for c in range(NCHUNK):
            gcp[c].wait()
            scp[c] = pltpu.async_copy(
                rows_v.at[c % 3], out_hbm.at[pl.ds(base + c * C, C)], ssem)
            nxt = c + 2
            if nxt < NCHUNK:
                if nxt - 3 >= 0:
                    scp[nxt - 3].wait()
                gcp[nxt] = pltpu.async_copy(
                    table_hbm.at[idx_v.at[nxt]], rows_v.at[nxt % 3], gsem)
        for c in range(max(0, NCHUNK - 3), NCHUNK):
            scp[c].wait()

    return _gather


def kernel(embeds, codebooks):
    idx3, gidx3, loss = _distances(embeds, codebooks)
    indices = idx3.reshape(N, H)
    gflat = gidx3.reshape(NW, NCHUNK, C)
    table = codebooks.reshape(H * K, D)
    qflat = _make_gather()(table, gflat)
    quantized = qflat.reshape(N, H, D)
    return quantized, indices, loss[0, 0]
